# baseline (speedup 1.0000x reference)
import functools

import jax
import jax.numpy as jnp
from jax import lax
from jax.experimental import pallas as pl
from jax.experimental.pallas import tpu as pltpu
from jax.experimental.pallas import tpu_sc as plsc

MEM = 100000
NQ = 512
C = 64
K = 5
BM = 2048
NFULL = MEM // BM
TAIL_OFF = NFULL * BM
TAILW = MEM - TAIL_OFF
NB = NFULL + 1
G = 16
CAR = 8


SCALE = 4096.0
QBIAS = 4200
IMAX = (1 << 17) - 1
MAGIC = 12582912.0


def _topk_body(xT_ref, mem_ref, w_ref, idx_ref, lin_ref, qn_ref, cs_ref):
    step = pl.program_id(0)

    @pl.when(step == 0)
    def _init():
        x = xT_ref[...]
        inv = lax.rsqrt(jnp.maximum(jnp.sum(x * x, axis=0, keepdims=True), 1e-24))
        qn_ref[...] = (x * (inv * SCALE)).astype(jnp.bfloat16)
        cs_ref[...] = jnp.zeros((CAR, NQ), jnp.float32)

    def _select(m, base, w):
        minv = lax.rsqrt(jnp.maximum(jnp.sum(m * m, axis=0, keepdims=True), 1e-24))
        mb = (m * minv).astype(jnp.bfloat16)
        simq = lax.dot_general(mb, qn_ref[...], (((0,), (0,)), ((), ())),
                               preferred_element_type=jnp.float32)
        rowneg = lax.broadcasted_iota(jnp.int32, (w, NQ), 0)
        cbase = (QBIAS << 17) + IMAX - base
        q = lax.bitcast_convert_type(simq + MAGIC, jnp.int32)
        packed = lax.shift_left(q, 17) + (cbase - rowneg)
        pf = lax.bitcast_convert_type(packed, jnp.float32)

        cmax = jnp.max(pf.reshape(w // G, G, NQ), axis=1)
        vals = jnp.concatenate([cs_ref[...], cmax], axis=0)
        top = []
        for _ in range(K):
            mx = jnp.max(vals, axis=0)
            top.append(mx)
            vals = jnp.where(vals == mx[None, :], 0.0, vals)
        for i in range(K):
            cs_ref[pl.ds(i, 1), :] = top[i][None, :]
        return top

    @pl.when(step < NFULL)
    def _main():
        m = mem_ref[:, pl.ds(step * BM, BM)]
        _select(m, step * BM, BM)
        mt = lax.transpose(m, (1, 0))
        lin_ref[...] = jnp.concatenate(
            [mt, jnp.zeros((BM, C), jnp.float32)], axis=1)

    @pl.when(step == NB - 1)
    def _final():
        m = mem_ref[:, TAIL_OFF:]
        top = _select(m, TAIL_OFF, TAILW)
        mt = lax.transpose(m, (1, 0))
        lin_ref[:TAILW, :] = jnp.concatenate(
            [mt, jnp.zeros((TAILW, C), jnp.float32)], axis=1)
        pis = [lax.bitcast_convert_type(top[i], jnp.int32) for i in range(K)]
        sq = [lax.shift_right_arithmetic(pi, 17).astype(jnp.float32) for pi in pis]
        es = [jnp.exp((sq[i] - sq[0]) * (1.0 / SCALE)) for i in range(K)]
        tot = es[0]
        for i in range(1, K):
            tot = tot + es[i]
        inv_tot = 0.5 / tot
        ws = jnp.concatenate(
            [(es[i] * inv_tot)[None, :] for i in range(K)]
            + [jnp.zeros((CAR - K, NQ), jnp.float32)], axis=0)
        iis = jnp.concatenate(
            [(IMAX - (pis[i] & IMAX))[None, :] for i in range(K)]
            + [jnp.zeros((CAR - K, NQ), jnp.int32)], axis=0)
        w_ref[...] = lax.transpose(ws, (1, 0))
        idx_ref[...] = lax.transpose(iis, (1, 0))


def _topk_call(xT, mem_t):
    return pl.pallas_call(
        _topk_body,
        grid=(NB,),
        in_specs=[
            pl.BlockSpec((C, NQ), lambda i: (0, 0)),
            pl.BlockSpec((C, MEM), lambda i: (0, 0)),
        ],
        out_specs=[
            pl.BlockSpec((NQ, CAR), lambda i: (0, 0)),
            pl.BlockSpec((NQ, CAR), lambda i: (0, 0)),
            pl.BlockSpec((BM, 2 * C), lambda i: (i, 0)),
        ],
        out_shape=[
            jax.ShapeDtypeStruct((NQ, CAR), jnp.float32),
            jax.ShapeDtypeStruct((NQ, CAR), jnp.int32),
            jax.ShapeDtypeStruct((NB * BM, 2 * C), jnp.float32),
        ],
        scratch_shapes=[
            pltpu.VMEM((C, NQ), jnp.bfloat16),
            pltpu.VMEM((CAR, NQ), jnp.float32),
        ],
    )(xT, mem_t)


@functools.cache
def _make_sc_combine():
    nc, ns = 2, 16
    nw = nc * ns
    bw = (NQ * CAR) // nw
    qw = NQ // nw
    mesh = plsc.VectorSubcoreMesh(core_axis_name="c", subcore_axis_name="s")

    @functools.partial(
        pl.kernel, mesh=mesh,
        compiler_params=pltpu.CompilerParams(use_tc_tiling_on_sc=False),
        out_type=jax.ShapeDtypeStruct((NQ, C), jnp.float32),
        scratch_types=[
            pltpu.VMEM((bw,), jnp.int32),
            pltpu.VMEM((bw, 16), jnp.float32),
            pltpu.VMEM((bw, 2 * C), jnp.float32),
            pltpu.VMEM((qw, C), jnp.float32),
            pltpu.SemaphoreType.DMA,
        ],
    )
    def _sc_combine(x_hbm, mem_hbm, idx_hbm, w_hbm, out_hbm,
                    idx_v, w_v, rows_v, x_v, sem):
        wid = lax.axis_index("s") * nc + lax.axis_index("c")
        pltpu.sync_copy(idx_hbm.at[pl.ds(wid * bw, bw)], idx_v)
        pltpu.sync_copy(w_hbm.at[pl.ds(wid * bw, bw)], w_v)
        pltpu.sync_copy(x_hbm.at[pl.ds(wid * qw, qw)], x_v)
        pltpu.async_copy(mem_hbm.at[idx_v], rows_v, sem).wait()
        for q in range(qw):
            ws = [w_v[q * CAR + k, :] for k in range(K)]
            for c in range(C // 16):
                sl = pl.ds(c * 16, 16)
                acc = x_v[q, sl]
                for k in range(K):
                    acc = acc + ws[k] * rows_v[q * CAR + k, sl]
                x_v[q, sl] = acc
        pltpu.sync_copy(x_v, out_hbm.at[pl.ds(wid * qw, qw)])

    return _sc_combine


def kernel(x, memory_mean):
    b, s, c = x.shape
    xf = x.reshape(b * s, c)
    w8, i8, lin = _topk_call(xf.T, memory_mean.T)
    kf = i8.reshape(-1)
    wx = jnp.broadcast_to(w8.reshape(-1)[:, None], (NQ * CAR, 16))
    out = _make_sc_combine()(xf, lin, kf, wx)
    return out.reshape(b, s, c)


# speedup vs baseline: 1.5461x; 1.5461x over previous
import functools

import jax
import jax.numpy as jnp
from jax import lax
from jax.experimental import pallas as pl
from jax.experimental.pallas import tpu as pltpu
from jax.experimental.pallas import tpu_sc as plsc

MEM = 100000
NQ = 512
C = 64
K = 5
BM = 2048
NFULL = MEM // BM
NB = NFULL + 1
LASTOFF = 97920
T32 = 32
G = 16
CAR = 8


SCALE = 4096.0
QBIAS = 4200
IMAX = (1 << 17) - 1
MAGIC = 12582912.0


def _topk_body(xT_ref, mem_ref, t32_ref, w_ref, idx_ref, lin_ref,
               qn_ref, cs_ref, buf_ref, sem):
    step = pl.program_id(0)

    def _win_copy(slot, win):
        off = jnp.where(win < NFULL, win * BM, LASTOFF)
        return pltpu.make_async_copy(
            mem_ref.at[:, pl.ds(off, BM)], buf_ref.at[slot], sem.at[slot])

    @pl.when(step == 0)
    def _init():
        _win_copy(0, 0).start()
        _win_copy(1, 1).start()
        x = xT_ref[...]
        inv = lax.rsqrt(jnp.maximum(jnp.sum(x * x, axis=0, keepdims=True), 1e-24))
        qn_ref[...] = (x * (inv * SCALE)).astype(jnp.bfloat16)
        cs_ref[...] = jnp.zeros((CAR, NQ), jnp.float32)

    slot = lax.rem(step, 2)
    _win_copy(slot, step).wait()

    def _select(m, base, w):
        minv = lax.rsqrt(jnp.maximum(jnp.sum(m * m, axis=0, keepdims=True), 1e-24))
        mb = (m * minv).astype(jnp.bfloat16)
        simq = lax.dot_general(mb, qn_ref[...], (((0,), (0,)), ((), ())),
                               preferred_element_type=jnp.float32)
        rowneg = lax.broadcasted_iota(jnp.int32, (w, NQ), 0)
        cbase = (QBIAS << 17) + IMAX - base
        q = lax.bitcast_convert_type(simq + MAGIC, jnp.int32)
        packed = lax.shift_left(q, 17) + (cbase - rowneg)
        pf = lax.bitcast_convert_type(packed, jnp.float32)

        cmax = jnp.max(pf.reshape(w // G, G, NQ), axis=1)
        vals = jnp.concatenate([cs_ref[...], cmax], axis=0)
        top = []
        for _ in range(K):
            mx = jnp.max(vals, axis=0)
            top.append(mx)
            vals = jnp.where(vals == mx[None, :], 0.0, vals)
        for i in range(K):
            cs_ref[pl.ds(i, 1), :] = top[i][None, :]
        return top

    @pl.when(step < NFULL)
    def _main():
        m = buf_ref[slot]
        _select(m, step * BM, BM)
        mt = lax.transpose(m, (1, 0))
        lin_ref[...] = jnp.concatenate(
            [mt, jnp.zeros((BM, C), jnp.float32)], axis=1)

    @pl.when(step == NB - 1)
    def _final():
        m = buf_ref[slot]
        _select(m, LASTOFF, BM)
        t32 = t32_ref[...]
        top = _select(t32, MEM - T32, T32)
        mt = lax.transpose(m, (1, 0))
        pad = NFULL * BM - LASTOFF
        lin_ref[: BM - pad, :] = jnp.concatenate(
            [mt[pad:], jnp.zeros((BM - pad, C), jnp.float32)], axis=1)
        t32t = lax.transpose(t32, (1, 0))
        lin_ref[BM - pad: BM - pad + T32, :] = jnp.concatenate(
            [t32t, jnp.zeros((T32, C), jnp.float32)], axis=1)
        pis = [lax.bitcast_convert_type(top[i], jnp.int32) for i in range(K)]
        sq = [lax.shift_right_arithmetic(pi, 17).astype(jnp.float32) for pi in pis]
        es = [jnp.exp((sq[i] - sq[0]) * (1.0 / SCALE)) for i in range(K)]
        tot = es[0]
        for i in range(1, K):
            tot = tot + es[i]
        inv_tot = 0.5 / tot
        zf = jnp.zeros((1, NQ), jnp.float32)
        zi = jnp.zeros((1, NQ), jnp.int32)
        for i in range(CAR):
            if i < K:
                w_ref[pl.ds(i, 1), :] = (es[i] * inv_tot)[None, :]
                idx_ref[pl.ds(i, 1), :] = (IMAX - (pis[i] & IMAX))[None, :]
            else:
                w_ref[pl.ds(i, 1), :] = zf
                idx_ref[pl.ds(i, 1), :] = zi

    @pl.when(step < NB - 2)
    def _prefetch():
        _win_copy(slot, step + 2).start()


def _topk_call(xT, mem_t, tail32):
    return pl.pallas_call(
        _topk_body,
        grid=(NB,),
        in_specs=[
            pl.BlockSpec((C, NQ), lambda i: (0, 0)),
            pl.BlockSpec(memory_space=pl.ANY),
            pl.BlockSpec((C, T32), lambda i: (0, 0)),
        ],
        out_specs=[
            pl.BlockSpec((CAR, NQ), lambda i: (0, 0)),
            pl.BlockSpec((CAR, NQ), lambda i: (0, 0)),
            pl.BlockSpec((BM, 2 * C), lambda i: (i, 0)),
        ],
        out_shape=[
            jax.ShapeDtypeStruct((CAR, NQ), jnp.float32),
            jax.ShapeDtypeStruct((CAR, NQ), jnp.int32),
            jax.ShapeDtypeStruct((NB * BM, 2 * C), jnp.float32),
        ],
        scratch_shapes=[
            pltpu.VMEM((C, NQ), jnp.bfloat16),
            pltpu.VMEM((CAR, NQ), jnp.float32),
            pltpu.VMEM((2, C, BM), jnp.float32),
            pltpu.SemaphoreType.DMA((2,)),
        ],
    )(xT, mem_t, tail32)


@functools.cache
def _make_sc_combine():
    nc, ns = 2, 16
    nw = nc * ns
    bw = (NQ * K) // nw
    qw = NQ // nw
    mesh = plsc.VectorSubcoreMesh(core_axis_name="c", subcore_axis_name="s")

    @functools.partial(
        pl.kernel, mesh=mesh,
        compiler_params=pltpu.CompilerParams(use_tc_tiling_on_sc=False),
        out_type=jax.ShapeDtypeStruct((NQ, C), jnp.float32),
        scratch_types=[
            pltpu.VMEM((bw,), jnp.int32),
            pltpu.VMEM((bw, 16), jnp.float32),
            pltpu.VMEM((bw, 2 * C), jnp.float32),
            pltpu.VMEM((qw, C), jnp.float32),
            pltpu.SemaphoreType.DMA,
        ],
    )
    def _sc_combine(x_hbm, mem_hbm, idx_hbm, w_hbm, out_hbm,
                    idx_v, w_v, rows_v, x_v, sem):
        wid = lax.axis_index("s") * nc + lax.axis_index("c")
        pltpu.sync_copy(idx_hbm.at[pl.ds(wid * bw, bw)], idx_v)
        pltpu.sync_copy(w_hbm.at[pl.ds(wid * bw, bw)], w_v)
        pltpu.sync_copy(x_hbm.at[pl.ds(wid * qw, qw)], x_v)
        pltpu.async_copy(mem_hbm.at[idx_v], rows_v, sem).wait()
        for q in range(qw):
            ws = [w_v[q * K + k, :] for k in range(K)]
            for c in range(C // 16):
                sl = pl.ds(c * 16, 16)
                acc = x_v[q, sl]
                for k in range(K):
                    acc = acc + ws[k] * rows_v[q * K + k, sl]
                x_v[q, sl] = acc
        pltpu.sync_copy(x_v, out_hbm.at[pl.ds(wid * qw, qw)])

    return _sc_combine


def kernel(x, memory_mean):
    b, s, c = x.shape
    xf = x.reshape(b * s, c)
    mem_t = memory_mean.T
    w8, i8, lin = _topk_call(xf.T, mem_t, mem_t[:, MEM - T32:])
    wf = w8[:K].T.reshape(-1)
    kf = i8[:K].T.reshape(-1)
    wx = jnp.broadcast_to(wf[:, None], (NQ * K, 16))
    out = _make_sc_combine()(xf, lin, kf, wx)
    return out.reshape(b, s, c)


# speedup vs baseline: 1.6191x; 1.0472x over previous
import functools

import jax
import jax.numpy as jnp
from jax import lax
from jax.experimental import pallas as pl
from jax.experimental.pallas import tpu as pltpu
from jax.experimental.pallas import tpu_sc as plsc

MEM = 100000
NQ = 512
C = 64
K = 5
BM = 4096
NFULL = MEM // BM
NB = NFULL + 1
LASTOFF = 95872
T32 = 32
G = 16
CAR = 8


SCALE = 4096.0
QBIAS = 4200
IMAX = (1 << 17) - 1
MAGIC = 12582912.0


def _topk_body(xT_ref, mem_ref, t32_ref, w_ref, idx_ref, lin_ref,
               qn_ref, cs_ref, buf_ref, sem):
    step = pl.program_id(0)

    def _win_copy(slot, win):
        off = jnp.where(win < NFULL, win * BM, LASTOFF)
        return pltpu.make_async_copy(
            mem_ref.at[:, pl.ds(off, BM)], buf_ref.at[slot], sem.at[slot])

    @pl.when(step == 0)
    def _init():
        _win_copy(0, 0).start()
        _win_copy(1, 1).start()
        x = xT_ref[...]
        inv = lax.rsqrt(jnp.maximum(jnp.sum(x * x, axis=0, keepdims=True), 1e-24))
        qn_ref[...] = (x * (inv * SCALE)).astype(jnp.bfloat16)
        cs_ref[...] = jnp.zeros((CAR, NQ), jnp.float32)

    slot = lax.rem(step, 2)
    _win_copy(slot, step).wait()

    def _select(m, base, w):
        minv = lax.rsqrt(jnp.maximum(jnp.sum(m * m, axis=0, keepdims=True), 1e-24))
        mb = (m * minv).astype(jnp.bfloat16)
        simq = lax.dot_general(mb, qn_ref[...], (((0,), (0,)), ((), ())),
                               preferred_element_type=jnp.float32)
        rowneg = lax.broadcasted_iota(jnp.int32, (w, NQ), 0)
        cbase = (QBIAS << 17) + IMAX - base
        q = lax.bitcast_convert_type(simq + MAGIC, jnp.int32)
        packed = lax.shift_left(q, 17) + (cbase - rowneg)
        pf = lax.bitcast_convert_type(packed, jnp.float32)

        cmax = jnp.max(pf.reshape(w // G, G, NQ), axis=1)
        vals = jnp.concatenate([cs_ref[...], cmax], axis=0)
        top = []
        for _ in range(K):
            mx = jnp.max(vals, axis=0)
            top.append(mx)
            vals = jnp.where(vals == mx[None, :], 0.0, vals)
        for i in range(K):
            cs_ref[pl.ds(i, 1), :] = top[i][None, :]
        return top

    @pl.when(step < NFULL)
    def _main():
        m = buf_ref[slot]
        _select(m, step * BM, BM)
        mt = lax.transpose(m, (1, 0))
        lin_ref[...] = jnp.concatenate(
            [mt, jnp.zeros((BM, C), jnp.float32)], axis=1)

    @pl.when(step == NB - 1)
    def _final():
        m = buf_ref[slot]
        _select(m, LASTOFF, BM)
        t32 = t32_ref[...]
        top = _select(t32, MEM - T32, T32)
        mt = lax.transpose(m, (1, 0))
        pad = NFULL * BM - LASTOFF
        lin_ref[: BM - pad, :] = jnp.concatenate(
            [mt[pad:], jnp.zeros((BM - pad, C), jnp.float32)], axis=1)
        t32t = lax.transpose(t32, (1, 0))
        lin_ref[BM - pad: BM - pad + T32, :] = jnp.concatenate(
            [t32t, jnp.zeros((T32, C), jnp.float32)], axis=1)
        pis = [lax.bitcast_convert_type(top[i], jnp.int32) for i in range(K)]
        sq = [lax.shift_right_arithmetic(pi, 17).astype(jnp.float32) for pi in pis]
        es = [jnp.exp((sq[i] - sq[0]) * (1.0 / SCALE)) for i in range(K)]
        tot = es[0]
        for i in range(1, K):
            tot = tot + es[i]
        inv_tot = 0.5 / tot
        zf = jnp.zeros((1, NQ), jnp.float32)
        zi = jnp.zeros((1, NQ), jnp.int32)
        for i in range(CAR):
            if i < K:
                w_ref[pl.ds(i, 1), :] = (es[i] * inv_tot)[None, :]
                idx_ref[pl.ds(i, 1), :] = (IMAX - (pis[i] & IMAX))[None, :]
            else:
                w_ref[pl.ds(i, 1), :] = zf
                idx_ref[pl.ds(i, 1), :] = zi

    @pl.when(step < NB - 2)
    def _prefetch():
        _win_copy(slot, step + 2).start()


def _topk_call(xT, mem_t, tail32):
    return pl.pallas_call(
        _topk_body,
        grid=(NB,),
        in_specs=[
            pl.BlockSpec((C, NQ), lambda i: (0, 0)),
            pl.BlockSpec(memory_space=pl.ANY),
            pl.BlockSpec((C, T32), lambda i: (0, 0)),
        ],
        out_specs=[
            pl.BlockSpec((CAR, NQ), lambda i: (0, 0)),
            pl.BlockSpec((CAR, NQ), lambda i: (0, 0)),
            pl.BlockSpec((BM, 2 * C), lambda i: (i, 0)),
        ],
        out_shape=[
            jax.ShapeDtypeStruct((CAR, NQ), jnp.float32),
            jax.ShapeDtypeStruct((CAR, NQ), jnp.int32),
            jax.ShapeDtypeStruct((NB * BM, 2 * C), jnp.float32),
        ],
        scratch_shapes=[
            pltpu.VMEM((C, NQ), jnp.bfloat16),
            pltpu.VMEM((CAR, NQ), jnp.float32),
            pltpu.VMEM((2, C, BM), jnp.float32),
            pltpu.SemaphoreType.DMA((2,)),
        ],
    )(xT, mem_t, tail32)


@functools.cache
def _make_sc_combine():
    nc, ns = 2, 16
    nw = nc * ns
    bw = (NQ * K) // nw
    qw = NQ // nw
    mesh = plsc.VectorSubcoreMesh(core_axis_name="c", subcore_axis_name="s")

    @functools.partial(
        pl.kernel, mesh=mesh,
        compiler_params=pltpu.CompilerParams(use_tc_tiling_on_sc=False),
        out_type=jax.ShapeDtypeStruct((NQ, C), jnp.float32),
        scratch_types=[
            pltpu.VMEM((bw,), jnp.int32),
            pltpu.VMEM((bw, 16), jnp.float32),
            pltpu.VMEM((bw, 2 * C), jnp.float32),
            pltpu.VMEM((qw, C), jnp.float32),
            pltpu.SemaphoreType.DMA,
        ],
    )
    def _sc_combine(x_hbm, mem_hbm, idx_hbm, w_hbm, out_hbm,
                    idx_v, w_v, rows_v, x_v, sem):
        wid = lax.axis_index("s") * nc + lax.axis_index("c")
        pltpu.sync_copy(idx_hbm.at[pl.ds(wid * bw, bw)], idx_v)
        pltpu.sync_copy(w_hbm.at[pl.ds(wid * bw, bw)], w_v)
        pltpu.sync_copy(x_hbm.at[pl.ds(wid * qw, qw)], x_v)
        pltpu.async_copy(mem_hbm.at[idx_v], rows_v, sem).wait()
        for q in range(qw):
            ws = [w_v[q * K + k, :] for k in range(K)]
            for c in range(C // 16):
                sl = pl.ds(c * 16, 16)
                acc = x_v[q, sl]
                for k in range(K):
                    acc = acc + ws[k] * rows_v[q * K + k, sl]
                x_v[q, sl] = acc
        pltpu.sync_copy(x_v, out_hbm.at[pl.ds(wid * qw, qw)])

    return _sc_combine


def kernel(x, memory_mean):
    b, s, c = x.shape
    xf = x.reshape(b * s, c)
    mem_t = memory_mean.T
    w8, i8, lin = _topk_call(xf.T, mem_t, mem_t[:, MEM - T32:])
    wf = w8[:K].T.reshape(-1)
    kf = i8[:K].T.reshape(-1)
    wx = jnp.broadcast_to(wf[:, None], (NQ * K, 16))
    out = _make_sc_combine()(xf, lin, kf, wx)
    return out.reshape(b, s, c)
